# Initial kernel scaffold; baseline (speedup 1.0000x reference)
import jax
import jax.numpy as jnp
from jax.experimental import pallas as pl


def kernel(ufeat, ifeat, edge_index):
    raise NotImplementedError("write your pallas kernel here")


# speedup vs baseline: 3.1501x; 3.1501x over previous
import functools

import jax
import jax.numpy as jnp
from jax import lax
from jax.experimental import pallas as pl
from jax.experimental.pallas import tpu as pltpu
from jax.experimental.pallas import tpu_sc as plsc

N_NODES = 10000
N_EDGES = 320000
D_FEAT = 128
LANES = 16

NUM_CORES = 2
NUM_SUBCORES = 16
NUM_WORKERS = NUM_CORES * NUM_SUBCORES
E_PER_W = N_EDGES // NUM_WORKERS
CHUNK = 400
NCHUNKS = E_PER_W // CHUNK

_mesh = plsc.VectorSubcoreMesh(core_axis_name="c", subcore_axis_name="s")

_GATHER_DNUMS = lax.GatherDimensionNumbers(
    offset_dims=(), collapsed_slice_dims=(0,), start_index_map=(0,))


def _lane_take(x, idx):
    return lax.gather(x, idx[:, None], _GATHER_DNUMS, (1,),
                      mode=lax.GatherScatterMode.PROMISE_IN_BOUNDS)


@functools.partial(
    pl.kernel,
    out_type=jax.ShapeDtypeStruct((N_EDGES,), jnp.float32),
    mesh=_mesh,
    scratch_types=[
        pltpu.VMEM((CHUNK,), jnp.int32),
        pltpu.VMEM((CHUNK,), jnp.int32),
        pltpu.VMEM((CHUNK, D_FEAT), jnp.float32),
        pltpu.VMEM((CHUNK, D_FEAT), jnp.float32),
        pltpu.VMEM((CHUNK,), jnp.float32),
        pltpu.SemaphoreType.DMA,
        pltpu.SemaphoreType.DMA,
    ],
)
def _u_dot_v(src_hbm, dst_hbm, ufeat_hbm, ifeat_hbm, out_hbm,
             src_v, dst_v, u_v, v_v, o_v, sem_u, sem_v):
    wid = lax.axis_index("s") * NUM_CORES + lax.axis_index("c")
    base = wid * E_PER_W

    def chunk_body(j, _):
        cbase = base + j * CHUNK
        pltpu.sync_copy(src_hbm.at[pl.ds(cbase, CHUNK)], src_v)
        pltpu.sync_copy(dst_hbm.at[pl.ds(cbase, CHUNK)], dst_v)
        cp_u = pltpu.async_copy(ufeat_hbm.at[src_v], u_v, sem_u)
        cp_v = pltpu.async_copy(ifeat_hbm.at[dst_v], v_v, sem_v)
        cp_u.wait()
        cp_v.wait()

        lane_iota = lax.iota(jnp.int32, LANES)
        perms = [(lane_iota ^ sh).astype(jnp.int32) for sh in (8, 4, 2, 1)]

        def group_body(g, _):
            e0 = g * LANES
            res = jnp.zeros((LANES,), jnp.float32)
            for k in range(LANES):
                e = e0 + k
                acc = u_v[e, pl.ds(0, LANES)] * v_v[e, pl.ds(0, LANES)]
                for b in range(1, D_FEAT // LANES):
                    acc = acc + (u_v[e, pl.ds(b * LANES, LANES)]
                                 * v_v[e, pl.ds(b * LANES, LANES)])
                for p in perms:
                    acc = acc + _lane_take(acc, p)
                res = jnp.where(lane_iota == k, acc, res)
            o_v[pl.ds(e0, LANES)] = res
            return 0

        lax.fori_loop(0, CHUNK // LANES, group_body, 0)
        pltpu.sync_copy(o_v, out_hbm.at[pl.ds(cbase, CHUNK)])
        return 0

    lax.fori_loop(0, NCHUNKS, chunk_body, 0)


def kernel(ufeat, ifeat, edge_index):
    src = edge_index[0].astype(jnp.int32)
    dst = edge_index[1].astype(jnp.int32)
    sr = _u_dot_v(src, dst, ufeat, ifeat)
    return (sr[:, None], ufeat, ifeat)
